# baseline (speedup 1.0000x reference)
import functools

import jax
import jax.numpy as jnp
from jax import lax
from jax.experimental import pallas as pl
from jax.experimental.pallas import tpu as pltpu
from jax.experimental.pallas import tpu_sc as plsc

EMBED_DIM = 64
SEQ_LEN = 200
BATCH = 1024
NUM_CORES = 2
NUM_SUBCORES = 16
NUM_WORKERS = NUM_CORES * NUM_SUBCORES
CHUNK = 128
TC_PER_L = BATCH // CHUNK
NBUF = 5
LEAD = 2
LANES = 16
LWIN = 7


def _sc_body(n_chunks, x_hbm, table_hbm, pe_hbm, out_hbm,
             idx_v, pe_v, rows_v, gsems, wsems):
    wid = lax.axis_index("s") * NUM_CORES + lax.axis_index("c")
    ci0 = wid * n_chunks
    l0 = ci0 // TC_PER_L

    pltpu.sync_copy(x_hbm.at[pl.ds(l0, LWIN)], idx_v)
    pltpu.sync_copy(pe_hbm.at[pl.ds(0, SEQ_LEN)], pe_v)

    def chunk_idx(j):
        ci = ci0 + j
        r = ci // TC_PER_L - l0
        tc = lax.rem(ci, TC_PER_L)
        return idx_v.at[r, pl.ds(tc * CHUNK, CHUNK)]

    def start_gather(j, b):
        pltpu.async_copy(table_hbm.at[chunk_idx(j)], rows_v.at[b], gsems.at[b])

    def wait_gather(j, b):
        pltpu.make_async_copy(
            table_hbm.at[chunk_idx(j)], rows_v.at[b], gsems.at[b]).wait()

    def start_write(j, b):
        ci = ci0 + j
        l = ci // TC_PER_L
        tc = lax.rem(ci, TC_PER_L)
        pltpu.async_copy(
            rows_v.at[b],
            out_hbm.at[pl.ds(tc * CHUNK, CHUNK), l],
            wsems.at[b])

    def wait_write(b):
        pltpu.make_async_copy(
            rows_v.at[b],
            out_hbm.at[pl.ds(0, CHUNK), 0],
            wsems.at[b]).wait()

    def compute(j, b):
        ci = ci0 + j
        l = ci // TC_PER_L
        pe_q = [pe_v[l, pl.ds(q * LANES, LANES)]
                for q in range(EMBED_DIM // LANES)]

        def add_row(i, carry2):
            for q in range(EMBED_DIM // LANES):
                sl = pl.ds(q * LANES, LANES)
                rows_v[b, i, sl] = rows_v[b, i, sl] + pe_q[q]
            return carry2

        lax.fori_loop(0, CHUNK, add_row, 0, unroll=4)

    for j in range(LEAD):
        start_gather(j, j % NBUF)

    def outer(j5, carry):
        for b in range(NBUF):
            j = j5 * NBUF + b
            bn = (b + LEAD) % NBUF

            @pl.when(j + LEAD < n_chunks)
            def _():
                @pl.when(j >= NBUF - LEAD)
                def _():
                    wait_write(bn)
                start_gather(j + LEAD, bn)

            wait_gather(j, b)
            compute(j, b)
            start_write(j, b)
        return carry

    lax.fori_loop(0, n_chunks // NBUF, outer, 0)

    for b in range(NBUF):
        wait_write(b)


def kernel(x, token_table, pe_table):
    B, L = x.shape
    total = B * L
    n_chunks = total // (NUM_WORKERS * CHUNK)
    assert n_chunks * NUM_WORKERS * CHUNK == total
    assert n_chunks % NBUF == 0 and NBUF > LEAD
    assert B % CHUNK == 0 and L == SEQ_LEN
    assert (n_chunks - 1) // TC_PER_L + 1 <= LWIN

    xt = x.T.astype(jnp.int32)
    pe_s = pe_table[:SEQ_LEN]

    mesh = plsc.VectorSubcoreMesh(core_axis_name="c", subcore_axis_name="s")
    run = pl.kernel(
        functools.partial(_sc_body, n_chunks),
        out_type=jax.ShapeDtypeStruct((B, L, EMBED_DIM), jnp.float32),
        mesh=mesh,
        compiler_params=pltpu.CompilerParams(use_tc_tiling_on_sc=False),
        scratch_types=[
            pltpu.VMEM((LWIN, BATCH), jnp.int32),
            pltpu.VMEM((SEQ_LEN, EMBED_DIM), jnp.float32),
            pltpu.VMEM((NBUF, CHUNK, EMBED_DIM), jnp.float32),
            pltpu.SemaphoreType.DMA((NBUF,)),
            pltpu.SemaphoreType.DMA((NBUF,)),
        ],
    )
    return run(xt, token_table, pe_s)


# speedup vs baseline: 1.0455x; 1.0455x over previous
import functools

import jax
import jax.numpy as jnp
from jax import lax
from jax.experimental import pallas as pl
from jax.experimental.pallas import tpu as pltpu
from jax.experimental.pallas import tpu_sc as plsc

EMBED_DIM = 64
SEQ_LEN = 200
BATCH = 1024
NUM_CORES = 2
NUM_SUBCORES = 16
NUM_WORKERS = NUM_CORES * NUM_SUBCORES
CHUNK = 128
TC_PER_L = BATCH // CHUNK
NBUF = 5
LEAD = 2
LANES = 16
LTILES = SEQ_LEN // 8


def _sc_body(n_chunks, x_hbm, table_hbm, pe_hbm, out_hbm,
             idx_v, pe_v, rows_v, gsems, wsems):
    wid = lax.axis_index("s") * NUM_CORES + lax.axis_index("c")
    ci0 = wid * n_chunks
    l0 = ci0 // TC_PER_L
    tl_base = lax.min(l0 // 8, LTILES - 2)

    pltpu.sync_copy(x_hbm.at[pl.ds(tl_base, 2)], idx_v)
    pltpu.sync_copy(pe_hbm.at[pl.ds(0, SEQ_LEN)], pe_v)

    def chunk_idx(j):
        ci = ci0 + j
        l = ci // TC_PER_L
        tc = lax.rem(ci, TC_PER_L)
        return idx_v.at[l // 8 - tl_base, tc, lax.rem(l, 8)]

    def start_gather(j, b):
        pltpu.async_copy(table_hbm.at[chunk_idx(j)], rows_v.at[b], gsems.at[b])

    def wait_gather(j, b):
        pltpu.make_async_copy(
            table_hbm.at[chunk_idx(j)], rows_v.at[b], gsems.at[b]).wait()

    def start_write(j, b):
        ci = ci0 + j
        l = ci // TC_PER_L
        tc = lax.rem(ci, TC_PER_L)
        pltpu.async_copy(
            rows_v.at[b],
            out_hbm.at[pl.ds(tc * CHUNK, CHUNK),
                       pl.ds(l * EMBED_DIM, EMBED_DIM)],
            wsems.at[b])

    def wait_write(b):
        pltpu.make_async_copy(
            rows_v.at[b],
            out_hbm.at[pl.ds(0, CHUNK), pl.ds(0, EMBED_DIM)],
            wsems.at[b]).wait()

    def compute(j, b):
        ci = ci0 + j
        l = ci // TC_PER_L
        pe_q = [pe_v[l, pl.ds(q * LANES, LANES)]
                for q in range(EMBED_DIM // LANES)]

        def add_row(i, carry2):
            for q in range(EMBED_DIM // LANES):
                sl = pl.ds(q * LANES, LANES)
                rows_v[b, i, sl] = rows_v[b, i, sl] + pe_q[q]
            return carry2

        lax.fori_loop(0, CHUNK, add_row, 0, unroll=4)

    for j in range(LEAD):
        start_gather(j, j % NBUF)

    def outer(j5, carry):
        for b in range(NBUF):
            j = j5 * NBUF + b
            bn = (b + LEAD) % NBUF

            @pl.when(j + LEAD < n_chunks)
            def _():
                @pl.when(j >= NBUF - LEAD)
                def _():
                    wait_write(bn)
                start_gather(j + LEAD, bn)

            wait_gather(j, b)
            compute(j, b)
            start_write(j, b)
        return carry

    lax.fori_loop(0, n_chunks // NBUF, outer, 0)

    for b in range(NBUF):
        wait_write(b)


def kernel(x, token_table, pe_table):
    B, L = x.shape
    total = B * L
    n_chunks = total // (NUM_WORKERS * CHUNK)
    assert n_chunks * NUM_WORKERS * CHUNK == total
    assert n_chunks % NBUF == 0 and NBUF > LEAD
    assert B % CHUNK == 0 and L == SEQ_LEN and L % 8 == 0

    xn = (x.astype(jnp.int32)
          .reshape(TC_PER_L, CHUNK, LTILES, 8)
          .transpose(2, 0, 3, 1))
    pe_s = pe_table[:SEQ_LEN]

    mesh = plsc.VectorSubcoreMesh(core_axis_name="c", subcore_axis_name="s")
    run = pl.kernel(
        functools.partial(_sc_body, n_chunks),
        out_type=jax.ShapeDtypeStruct((B, L * EMBED_DIM), jnp.float32),
        mesh=mesh,
        compiler_params=pltpu.CompilerParams(use_tc_tiling_on_sc=False),
        scratch_types=[
            pltpu.VMEM((2, TC_PER_L, 8, CHUNK), jnp.int32),
            pltpu.VMEM((SEQ_LEN, EMBED_DIM), jnp.float32),
            pltpu.VMEM((NBUF, CHUNK, EMBED_DIM), jnp.float32),
            pltpu.SemaphoreType.DMA((NBUF,)),
            pltpu.SemaphoreType.DMA((NBUF,)),
        ],
    )
    out = run(xn, token_table, pe_s)
    return out.reshape(B, L, EMBED_DIM)
